# baseline (speedup 1.0000x reference)
import functools

import jax
import jax.numpy as jnp
from jax import lax
from jax.experimental import pallas as pl
from jax.experimental.pallas import tpu as pltpu
from jax.experimental.pallas import tpu_sc as plsc

F = 26
V = 100000
D = 16
B = 16384
TOTAL = B * F
NC, NS, L = 2, 16, 16
NW = NC * NS
PER_W = TOTAL // NW
CHUNK = 1664
NCH = PER_W // CHUNK
GSZ = 128
NG = CHUNK // GSZ

_mesh = plsc.VectorSubcoreMesh(core_axis_name="c", subcore_axis_name="s")


@functools.partial(
    pl.kernel,
    mesh=_mesh,
    compiler_params=pltpu.CompilerParams(use_tc_tiling_on_sc=False),
    out_type=jax.ShapeDtypeStruct((TOTAL, D), jnp.float32),
    scratch_types=[
        pltpu.VMEM((CHUNK,), jnp.int32),
        pltpu.VMEM((CHUNK,), jnp.int32),
        pltpu.VMEM((CHUNK, D), jnp.float32),
        pltpu.SemaphoreType.DMA,
    ],
)
def _gather_kernel(x_hbm, off_hbm, table_hbm, out_hbm, idx_v, off_v, rows_v, sem):
    wid = lax.axis_index("s") * NC + lax.axis_index("c")
    base = wid * PER_W

    pltpu.sync_copy(off_hbm, off_v)

    for c in range(NCH):
        start = base + c * CHUNK
        pltpu.sync_copy(x_hbm.at[pl.ds(start, CHUNK)], idx_v)

        def add_body(k, carry):
            s = pl.ds(k * L, L)
            idx_v[s] = idx_v[s] + off_v[s]
            return carry

        lax.fori_loop(0, CHUNK // L, add_body, 0)

        descs = []
        for g in range(NG):
            s = pl.ds(g * GSZ, GSZ)
            descs.append(
                pltpu.async_copy(table_hbm.at[idx_v.at[s]], rows_v.at[s], sem)
            )
        for d in descs:
            d.wait()

        pltpu.sync_copy(rows_v, out_hbm.at[pl.ds(start, CHUNK)])


def kernel(x, tables):
    x_flat = x.astype(jnp.int32).reshape(TOTAL)
    flat_tables = tables.reshape(F * V, D)
    off = jnp.tile(jnp.arange(F, dtype=jnp.int32) * V, CHUNK // F)
    out = _gather_kernel(x_flat, off, flat_tables)
    return out.reshape(B, F * D)
